# baseline (speedup 1.0000x reference)
import functools

import jax
import jax.numpy as jnp
from jax import lax
from jax.experimental import pallas as pl
from jax.experimental.pallas import tpu as pltpu
import jax.experimental.pallas.tpu_sc as plsc

N_NODES = 10000
N_EDGES = 320000
NC = 2
NS = 16
LANES = 16
FC = 128
SLAB = 1000
ZROWS = 40
K = 80
NB = 4

_MESH = dict(core_axis_name="c", subcore_axis_name="s")


def _zero_slab(zbuf, acc, s):
    my0 = s * SLAB

    @pl.when(s < N_NODES // SLAB)
    def _zero():
        for t in range(SLAB // ZROWS):
            pltpu.sync_copy(zbuf, acc.at[pl.ds(my0 + t * ZROWS, ZROWS)])

    plsc.subcore_barrier()


def _writeout_slab(acc, out_hbm, s, out_off):
    plsc.subcore_barrier()
    my0 = s * SLAB

    @pl.when(s < N_NODES // SLAB)
    def _writeout():
        pltpu.sync_copy(acc.at[pl.ds(my0, SLAB)],
                        out_hbm.at[pl.ds(out_off + my0, SLAB)])


def _fill(buf, nrows, value):
    vv = jnp.full((LANES,), value, jnp.float32)

    def body(r, carry):
        for j in range(FC // LANES):
            buf[r, pl.ds(j * LANES, LANES)] = vv
        return carry

    lax.fori_loop(0, nrows, body, 0)


def _maybe(cond, fn):
    if isinstance(cond, bool):
        if cond:
            fn()
    else:
        pl.when(cond)(fn)


def _sc_degree_body(dst_hbm, out_hbm, idxd, ones_b, zbuf, acc,
                    i0, i1, i2, i3, t0, t1, t2, t3):
    c = lax.axis_index("c")
    s = lax.axis_index("s")
    _fill(ones_b, K, 1.0)
    _fill(zbuf, ZROWS, 0.0)
    _zero_slab(zbuf, acc, s)

    e_per = N_EDGES // (NC * NS)
    base = (c * NS + s) * e_per
    nch = e_per // K
    isems = (i0, i1, i2, i3)
    ssems = (t0, t1, t2, t3)

    def fire_i(g, b):
        pltpu.async_copy(dst_hbm.at[pl.ds(base + g * K, K)], idxd.at[b],
                         isems[b])

    def wait_i(b):
        pltpu.make_async_copy(dst_hbm.at[pl.ds(base, K)], idxd.at[b],
                              isems[b]).wait()

    def fire_s(g, b):
        pltpu.async_copy(ones_b, acc.at[idxd.at[b]], ssems[b], add=True)

    def wait_s(b):
        pltpu.make_async_copy(ones_b, acc.at[idxd.at[b]], ssems[b]).wait()

    def slot(g, j):
        wait_i(j)
        jp = (j + 3) % NB
        _maybe(g - 1 >= 0 if isinstance(g, int) else g >= 1,
               lambda: wait_s(jp))
        fire_s(g, j)
        _maybe(g + 3 < nch, lambda: fire_i(g + 3, jp))

    for g in range(3):
        fire_i(g, g)

    def quad(i, carry):
        for j in range(NB):
            slot(i * NB + j, j)
        return carry

    nquad = nch // NB
    lax.fori_loop(0, nquad, quad, 0)
    for g in range(nquad * NB, nch):
        slot(g, g % NB)
    wait_s((nch - 1) % NB)
    _writeout_slab(acc, out_hbm, s, c * N_NODES)


def _make_sc_degree():
    return functools.partial(
        pl.kernel,
        out_type=jax.ShapeDtypeStruct((NC * N_NODES, FC), jnp.float32),
        mesh=plsc.VectorSubcoreMesh(**_MESH),
        scratch_types=[
            pltpu.VMEM((NB, K), jnp.int32),
            pltpu.VMEM((K, FC), jnp.float32),
            pltpu.VMEM((ZROWS, FC), jnp.float32),
            pltpu.VMEM_SHARED((N_NODES, FC), jnp.float32),
        ] + [pltpu.SemaphoreType.DMA] * 8,
    )(_sc_degree_body)


def _sc_agg_body(split_edges, src_hbm, dst_hbm, y_hbm, out_hbm,
                 idxg, idxd, rows, zbuf, acc,
                 i0, i1, i2, i3, g0, g1, g2, g3, t0, t1, t2, t3):
    c = lax.axis_index("c")
    s = lax.axis_index("s")
    _fill(zbuf, ZROWS, 0.0)
    _zero_slab(zbuf, acc, s)

    if split_edges:
        e_per = N_EDGES // (NC * NS)
        base = (c * NS + s) * e_per
        off = None
    else:
        e_per = N_EDGES // NS
        base = s * e_per
        off = c * N_NODES
    nch = e_per // K
    isems = (i0, i1, i2, i3)
    gsems = (g0, g1, g2, g3)
    ssems = (t0, t1, t2, t3)

    def fire_i(g, b):
        eb = base + g * K
        pltpu.async_copy(src_hbm.at[pl.ds(eb, K)], idxg.at[b], isems[b])
        pltpu.async_copy(dst_hbm.at[pl.ds(eb, K)], idxd.at[b], isems[b])

    def wait_i(b):
        pltpu.make_async_copy(src_hbm.at[pl.ds(base, K)], idxg.at[b],
                              isems[b]).wait()
        pltpu.make_async_copy(dst_hbm.at[pl.ds(base, K)], idxd.at[b],
                              isems[b]).wait()
        if off is not None:
            for j in range(K // LANES):
                v = idxg[b, pl.ds(j * LANES, LANES)]
                idxg[b, pl.ds(j * LANES, LANES)] = v + off

    def fire_g(g, b):
        pltpu.async_copy(y_hbm.at[idxg.at[b]], rows.at[b], gsems[b])

    def wait_g(b):
        pltpu.make_async_copy(y_hbm.at[idxg.at[b]], rows.at[b],
                              gsems[b]).wait()

    def fire_s(g, b):
        pltpu.async_copy(rows.at[b], acc.at[idxd.at[b]], ssems[b], add=True)

    def wait_s(b):
        pltpu.make_async_copy(rows.at[b], acc.at[idxd.at[b]],
                              ssems[b]).wait()

    def slot(g, j):
        wait_g(j)
        jp = (j + 3) % NB
        _maybe(g - 1 >= 0 if isinstance(g, int) else g >= 1,
               lambda: wait_s(jp))
        fire_s(g, j)
        jn = (j + 2) % NB

        def _adv():
            wait_i(jn)
            fire_g(g + 2, jn)

        _maybe(g + 2 < nch, _adv)
        _maybe(g + 3 < nch, lambda: fire_i(g + 3, jp))

    for g in range(3):
        fire_i(g, g)
    wait_i(0)
    fire_g(0, 0)
    wait_i(1)
    fire_g(1, 1)

    def quad(i, carry):
        for j in range(NB):
            slot(i * NB + j, j)
        return carry

    nquad = nch // NB
    lax.fori_loop(0, nquad, quad, 0)
    for g in range(nquad * NB, nch):
        slot(g, g % NB)
    wait_s((nch - 1) % NB)
    _writeout_slab(acc, out_hbm, s, c * N_NODES)


def _make_sc_agg(split_edges):
    return functools.partial(
        pl.kernel,
        out_type=jax.ShapeDtypeStruct((NC * N_NODES, FC), jnp.float32),
        mesh=plsc.VectorSubcoreMesh(**_MESH),
        scratch_types=[
            pltpu.VMEM((NB, K), jnp.int32),
            pltpu.VMEM((NB, K), jnp.int32),
            pltpu.VMEM((NB, K, FC), jnp.float32),
            pltpu.VMEM((ZROWS, FC), jnp.float32),
            pltpu.VMEM_SHARED((N_NODES, FC), jnp.float32),
        ] + [pltpu.SemaphoreType.DMA] * 12,
    )(functools.partial(_sc_agg_body, split_edges))


def _dinv_from_parts(degp):
    deg = degp[0:N_NODES, 0:1] + degp[N_NODES:2 * N_NODES, 0:1] + 1.0
    return lax.rsqrt(deg)


def _tc_matmul1_body(x_ref, w_ref, xw_ref):
    xw_ref[...] = jnp.dot(x_ref[...], w_ref[...],
                          preferred_element_type=jnp.float32)


def _tc_scale1_body(xw_ref, degp_ref, y_ref):
    dinv = _dinv_from_parts(degp_ref[...])
    y_ref[...] = xw_ref[...] * dinv


def _tc_layer2_body(agg1_ref, y1_ref, w2_ref, b1_ref, degp_ref, y2_ref):
    dinv = _dinv_from_parts(degp_ref[...])
    agg = agg1_ref[0:N_NODES, :] + agg1_ref[N_NODES:2 * N_NODES, :]
    out1 = (agg + y1_ref[...]) * dinv + b1_ref[...]
    h = jnp.maximum(out1, 0.0)
    y2 = jnp.dot(h, w2_ref[...], preferred_element_type=jnp.float32) * dinv
    hw = y2.shape[1] // 2
    y2_ref[0:N_NODES, :] = y2[:, 0:hw]
    y2_ref[N_NODES:2 * N_NODES, :] = y2[:, hw:]


def _tc_final_body(agg2_ref, y2_ref, b2_ref, degp_ref, batch_ref, wg_ref,
                   bg_ref, emb_ref, logit_ref):
    dinv = _dinv_from_parts(degp_ref[...])
    lo = agg2_ref[0:N_NODES, :] + y2_ref[0:N_NODES, :]
    hi = agg2_ref[N_NODES:2 * N_NODES, :] + y2_ref[N_NODES:2 * N_NODES, :]
    out2 = jnp.concatenate([lo, hi], axis=1) * dinv + b2_ref[...]
    ngr = emb_ref.shape[0]
    seg = batch_ref[...]
    p = (seg == lax.broadcasted_iota(jnp.int32, (1, ngr), 1)).astype(
        jnp.float32)
    sums = lax.dot_general(p, out2, (((0,), (0,)), ((), ())),
                           preferred_element_type=jnp.float32)
    counts = lax.dot_general(p, jnp.ones((N_NODES, 1), jnp.float32),
                             (((0,), (0,)), ((), ())),
                             preferred_element_type=jnp.float32)
    emb = sums / jnp.maximum(counts, 1.0)
    emb_ref[...] = emb
    logit_ref[...] = (jnp.dot(emb, wg_ref[...],
                              preferred_element_type=jnp.float32)
                      + bg_ref[...])


def kernel(x, edge_index, batch, W1, b1, W2, b2, Wg, bg):
    n, _ = x.shape
    hid = W1.shape[1]
    emb_d = W2.shape[1]
    ngroups = Wg.shape[1]
    ngraphs = 64
    src = edge_index[0]
    dst = edge_index[1]

    degp = _make_sc_degree()(dst)

    xw1 = pl.pallas_call(
        _tc_matmul1_body,
        out_shape=jax.ShapeDtypeStruct((n, hid), jnp.float32),
    )(x, W1)

    y1 = pl.pallas_call(
        _tc_scale1_body,
        out_shape=jax.ShapeDtypeStruct((n, hid), jnp.float32),
    )(xw1, degp)

    agg1 = _make_sc_agg(split_edges=True)(src, dst, y1)

    y2cat = pl.pallas_call(
        _tc_layer2_body,
        out_shape=jax.ShapeDtypeStruct((2 * n, emb_d // 2), jnp.float32),
    )(agg1, y1, W2, b1.reshape(1, -1), degp)

    agg2 = _make_sc_agg(split_edges=False)(src, dst, y2cat)

    emb, logits = pl.pallas_call(
        _tc_final_body,
        out_shape=(
            jax.ShapeDtypeStruct((ngraphs, emb_d), jnp.float32),
            jax.ShapeDtypeStruct((ngraphs, ngroups), jnp.float32),
        ),
    )(agg2, y2cat, b2.reshape(1, -1), degp, batch.reshape(-1, 1), Wg,
      bg.reshape(1, -1))

    return emb, logits


# speedup vs baseline: 1.2548x; 1.2548x over previous
import functools

import jax
import jax.numpy as jnp
from jax import lax
from jax.experimental import pallas as pl
from jax.experimental.pallas import tpu as pltpu
import jax.experimental.pallas.tpu_sc as plsc

N_NODES = 10000
N_EDGES = 320000
NC = 2
NS = 16
LANES = 16
FC = 128
SLAB = 1000
ZROWS = 40
K = 80
NB = 4

_MESH = dict(core_axis_name="c", subcore_axis_name="s")


def _zero_slab(zbuf, acc, s):
    my0 = s * SLAB

    @pl.when(s < N_NODES // SLAB)
    def _zero():
        for t in range(SLAB // ZROWS):
            pltpu.sync_copy(zbuf, acc.at[pl.ds(my0 + t * ZROWS, ZROWS)])

    plsc.subcore_barrier()


def _writeout_slab(acc, out_hbm, s, out_off):
    plsc.subcore_barrier()
    my0 = s * SLAB

    @pl.when(s < N_NODES // SLAB)
    def _writeout():
        pltpu.sync_copy(acc.at[pl.ds(my0, SLAB)],
                        out_hbm.at[pl.ds(out_off + my0, SLAB)])


def _fill(buf, nrows, value):
    vv = jnp.full((LANES,), value, jnp.float32)

    def body(r, carry):
        for j in range(FC // LANES):
            buf[r, pl.ds(j * LANES, LANES)] = vv
        return carry

    lax.fori_loop(0, nrows, body, 0)


def _maybe(cond, fn):
    if isinstance(cond, bool):
        if cond:
            fn()
    else:
        pl.when(cond)(fn)


def _sc_degree_body(dst_hbm, out_hbm, idxd, ones_b, zbuf, acc,
                    i0, i1, i2, i3, t0, t1, t2, t3):
    c = lax.axis_index("c")
    s = lax.axis_index("s")
    _fill(ones_b, K, 1.0)
    _fill(zbuf, ZROWS, 0.0)
    _zero_slab(zbuf, acc, s)

    e_per = N_EDGES // (NC * NS)
    base = (c * NS + s) * e_per
    nch = e_per // K
    isems = (i0, i1, i2, i3)
    ssems = (t0, t1, t2, t3)

    def fire_i(g, b):
        pltpu.async_copy(dst_hbm.at[pl.ds(base + g * K, K)], idxd.at[b],
                         isems[b])

    def wait_i(b):
        pltpu.make_async_copy(dst_hbm.at[pl.ds(base, K)], idxd.at[b],
                              isems[b]).wait()

    def fire_s(g, b):
        pltpu.async_copy(ones_b, acc.at[idxd.at[b]], ssems[b], add=True)

    def wait_s(b):
        pltpu.make_async_copy(ones_b, acc.at[idxd.at[b]], ssems[b]).wait()

    def slot(g, j):
        wait_i(j)
        jp = (j + 3) % NB
        _maybe(g - 1 >= 0 if isinstance(g, int) else g >= 1,
               lambda: wait_s(jp))
        fire_s(g, j)
        _maybe(g + 3 < nch, lambda: fire_i(g + 3, jp))

    for g in range(3):
        fire_i(g, g)

    def quad(i, carry):
        for j in range(NB):
            slot(i * NB + j, j)
        return carry

    nquad = nch // NB
    lax.fori_loop(0, nquad, quad, 0)
    for g in range(nquad * NB, nch):
        slot(g, g % NB)
    wait_s((nch - 1) % NB)
    _writeout_slab(acc, out_hbm, s, c * N_NODES)


def _make_sc_degree():
    return functools.partial(
        pl.kernel,
        out_type=jax.ShapeDtypeStruct((NC * N_NODES, FC), jnp.float32),
        mesh=plsc.VectorSubcoreMesh(**_MESH),
        scratch_types=[
            pltpu.VMEM((NB, K), jnp.int32),
            pltpu.VMEM((K, FC), jnp.float32),
            pltpu.VMEM((ZROWS, FC), jnp.float32),
            pltpu.VMEM_SHARED((N_NODES, FC), jnp.float32),
        ] + [pltpu.SemaphoreType.DMA] * 8,
    )(_sc_degree_body)


def _sc_agg_body(split_edges, src_hbm, dst_hbm, y_hbm, out_hbm,
                 idxg, idxd, rows, zbuf, acc,
                 i0, i1, i2, i3, g0, g1, g2, g3, t0, t1, t2, t3):
    c = lax.axis_index("c")
    s = lax.axis_index("s")
    _fill(zbuf, ZROWS, 0.0)
    _zero_slab(zbuf, acc, s)

    if split_edges:
        e_per = N_EDGES // (NC * NS)
        base = (c * NS + s) * e_per
        off = None
    else:
        e_per = N_EDGES // NS
        base = s * e_per
        off = c * N_NODES
    nch = e_per // K
    isems = (i0, i1, i2, i3)
    gsems = (g0, g1, g2, g3)
    ssems = (t0, t1, t2, t3)

    def fire_i(g, b):
        eb = base + g * K
        pltpu.async_copy(src_hbm.at[pl.ds(eb, K)], idxg.at[b], isems[b])
        pltpu.async_copy(dst_hbm.at[pl.ds(eb, K)], idxd.at[b], isems[b])

    def wait_i(b):
        pltpu.make_async_copy(src_hbm.at[pl.ds(base, K)], idxg.at[b],
                              isems[b]).wait()
        pltpu.make_async_copy(dst_hbm.at[pl.ds(base, K)], idxd.at[b],
                              isems[b]).wait()
        if off is not None:
            for j in range(K // LANES):
                v = idxg[b, pl.ds(j * LANES, LANES)]
                idxg[b, pl.ds(j * LANES, LANES)] = v + off

    def fire_g(g, b):
        pltpu.async_copy(y_hbm.at[idxg.at[b]], rows.at[b], gsems[b])

    def wait_g(b):
        pltpu.make_async_copy(y_hbm.at[idxg.at[b]], rows.at[b],
                              gsems[b]).wait()

    def fire_s(g, b):
        pltpu.async_copy(rows.at[b], acc.at[idxd.at[b]], ssems[b], add=True)

    def wait_s(b):
        pltpu.make_async_copy(rows.at[b], acc.at[idxd.at[b]],
                              ssems[b]).wait()

    def slot(g, j):
        wait_g(j)
        jp = (j + 3) % NB
        _maybe(g - 1 >= 0 if isinstance(g, int) else g >= 1,
               lambda: wait_s(jp))
        fire_s(g, j)
        jn = (j + 2) % NB

        def _adv():
            wait_i(jn)
            fire_g(g + 2, jn)

        _maybe(g + 2 < nch, _adv)
        _maybe(g + 3 < nch, lambda: fire_i(g + 3, jp))

    for g in range(3):
        fire_i(g, g)
    wait_i(0)
    fire_g(0, 0)
    wait_i(1)
    fire_g(1, 1)

    def quad(i, carry):
        for j in range(NB):
            slot(i * NB + j, j)
        return carry

    nquad = nch // NB
    lax.fori_loop(0, nquad, quad, 0)
    for g in range(nquad * NB, nch):
        slot(g, g % NB)
    wait_s((nch - 1) % NB)
    _writeout_slab(acc, out_hbm, s, c * N_NODES)


def _make_sc_agg(split_edges):
    return functools.partial(
        pl.kernel,
        out_type=jax.ShapeDtypeStruct((NC * N_NODES, FC), jnp.float32),
        mesh=plsc.VectorSubcoreMesh(**_MESH),
        scratch_types=[
            pltpu.VMEM((NB, K), jnp.int32),
            pltpu.VMEM((NB, K), jnp.int32),
            pltpu.VMEM((NB, K, FC), jnp.float32),
            pltpu.VMEM((ZROWS, FC), jnp.float32),
            pltpu.VMEM_SHARED((N_NODES, FC), jnp.float32),
        ] + [pltpu.SemaphoreType.DMA] * 12,
    )(functools.partial(_sc_agg_body, split_edges))


def _dinv_from_parts(degp):
    deg = degp[0:N_NODES, 0:1] + degp[N_NODES:2 * N_NODES, 0:1] + 1.0
    return lax.rsqrt(deg)


def _tc_matmul1_body(x_ref, w_ref, xw_ref):
    xw_ref[...] = jnp.dot(x_ref[...], w_ref[...],
                          preferred_element_type=jnp.float32)


def _tc_scale1_body(xw_ref, degp_ref, y_ref):
    dinv = _dinv_from_parts(degp_ref[...])
    y_ref[...] = xw_ref[...] * dinv


def _tc_layer2_body(agg1_ref, y1_ref, b1_ref, degp_ref, z_ref):
    dinv = _dinv_from_parts(degp_ref[...])
    agg = agg1_ref[0:N_NODES, :] + agg1_ref[N_NODES:2 * N_NODES, :]
    out1 = (agg + y1_ref[...]) * dinv + b1_ref[...]
    z_ref[...] = jnp.maximum(out1, 0.0) * dinv


def _tc_final_body(agg2_ref, z_ref, w2_ref, b2_ref, degp_ref, batch_ref,
                   wg_ref, bg_ref, emb_ref, logit_ref):
    dinv = _dinv_from_parts(degp_ref[...])
    agg = agg2_ref[0:N_NODES, :] + agg2_ref[N_NODES:2 * N_NODES, :]
    m = (agg + z_ref[...]) * dinv
    ngr = emb_ref.shape[0]
    seg = batch_ref[...]
    p = (seg == lax.broadcasted_iota(jnp.int32, (1, ngr), 1)).astype(
        jnp.float32)
    pooled = lax.dot_general(p, m, (((0,), (0,)), ((), ())),
                             preferred_element_type=jnp.float32)
    counts = lax.dot_general(p, jnp.ones((N_NODES, 1), jnp.float32),
                             (((0,), (0,)), ((), ())),
                             preferred_element_type=jnp.float32)
    sums = (jnp.dot(pooled, w2_ref[...], preferred_element_type=jnp.float32)
            + counts * b2_ref[...])
    emb = sums / jnp.maximum(counts, 1.0)
    emb_ref[...] = emb
    logit_ref[...] = (jnp.dot(emb, wg_ref[...],
                              preferred_element_type=jnp.float32)
                      + bg_ref[...])


def kernel(x, edge_index, batch, W1, b1, W2, b2, Wg, bg):
    n, _ = x.shape
    hid = W1.shape[1]
    emb_d = W2.shape[1]
    ngroups = Wg.shape[1]
    ngraphs = 64
    src = edge_index[0]
    dst = edge_index[1]

    degp = _make_sc_degree()(dst)

    xw1 = pl.pallas_call(
        _tc_matmul1_body,
        out_shape=jax.ShapeDtypeStruct((n, hid), jnp.float32),
    )(x, W1)

    y1 = pl.pallas_call(
        _tc_scale1_body,
        out_shape=jax.ShapeDtypeStruct((n, hid), jnp.float32),
    )(xw1, degp)

    agg1 = _make_sc_agg(split_edges=True)(src, dst, y1)

    z = pl.pallas_call(
        _tc_layer2_body,
        out_shape=jax.ShapeDtypeStruct((n, hid), jnp.float32),
    )(agg1, y1, b1.reshape(1, -1), degp)

    agg2 = _make_sc_agg(split_edges=True)(src, dst, z)

    emb, logits = pl.pallas_call(
        _tc_final_body,
        out_shape=(
            jax.ShapeDtypeStruct((ngraphs, emb_d), jnp.float32),
            jax.ShapeDtypeStruct((ngraphs, ngroups), jnp.float32),
        ),
    )(agg2, z, W2, b2.reshape(1, -1), degp, batch.reshape(-1, 1), Wg,
      bg.reshape(1, -1))

    return emb, logits


# speedup vs baseline: 1.2570x; 1.0018x over previous
import functools

import jax
import jax.numpy as jnp
from jax import lax
from jax.experimental import pallas as pl
from jax.experimental.pallas import tpu as pltpu
import jax.experimental.pallas.tpu_sc as plsc

N_NODES = 10000
N_EDGES = 320000
NC = 2
NS = 16
LANES = 16
FC = 128
SLAB = 1000
ZROWS = 40
K = 80
NB = 4

_MESH = dict(core_axis_name="c", subcore_axis_name="s")


def _zero_slab(zbuf, acc, s):
    my0 = s * SLAB

    @pl.when(s < N_NODES // SLAB)
    def _zero():
        for t in range(SLAB // ZROWS):
            pltpu.sync_copy(zbuf, acc.at[pl.ds(my0 + t * ZROWS, ZROWS)])

    plsc.subcore_barrier()


def _writeout_slab(acc, out_hbm, s, out_off):
    plsc.subcore_barrier()
    my0 = s * SLAB

    @pl.when(s < N_NODES // SLAB)
    def _writeout():
        pltpu.sync_copy(acc.at[pl.ds(my0, SLAB)],
                        out_hbm.at[pl.ds(out_off + my0, SLAB)])


def _fill(buf, nrows, value):
    vv = jnp.full((LANES,), value, jnp.float32)

    def body(r, carry):
        for j in range(FC // LANES):
            buf[r, pl.ds(j * LANES, LANES)] = vv
        return carry

    lax.fori_loop(0, nrows, body, 0)


def _maybe(cond, fn):
    if isinstance(cond, bool):
        if cond:
            fn()
    else:
        pl.when(cond)(fn)


def _sc_degree_body(dst_hbm, out_hbm, idxd, ones_b, zbuf, acc,
                    i0, i1, i2, i3, t0, t1, t2, t3):
    c = lax.axis_index("c")
    s = lax.axis_index("s")
    _fill(ones_b, K, 1.0)
    _fill(zbuf, ZROWS, 0.0)
    _zero_slab(zbuf, acc, s)

    e_per = N_EDGES // (NC * NS)
    base = (c * NS + s) * e_per
    nch = e_per // K
    isems = (i0, i1, i2, i3)
    ssems = (t0, t1, t2, t3)

    def fire_i(g, b):
        pltpu.async_copy(dst_hbm.at[pl.ds(base + g * K, K)], idxd.at[b],
                         isems[b])

    def wait_i(b):
        pltpu.make_async_copy(dst_hbm.at[pl.ds(base, K)], idxd.at[b],
                              isems[b]).wait()

    def fire_s(g, b):
        pltpu.async_copy(ones_b, acc.at[idxd.at[b]], ssems[b], add=True)

    def wait_s(b):
        pltpu.make_async_copy(ones_b, acc.at[idxd.at[b]], ssems[b]).wait()

    def slot(g, j):
        wait_i(j)
        jp = (j + 3) % NB
        _maybe(g - 1 >= 0 if isinstance(g, int) else g >= 1,
               lambda: wait_s(jp))
        fire_s(g, j)
        _maybe(g + 3 < nch, lambda: fire_i(g + 3, jp))

    for g in range(3):
        fire_i(g, g)

    def quad(i, carry):
        for j in range(NB):
            slot(i * NB + j, j)
        return carry

    nquad = nch // NB
    lax.fori_loop(0, nquad, quad, 0)
    for g in range(nquad * NB, nch):
        slot(g, g % NB)
    wait_s((nch - 1) % NB)
    _writeout_slab(acc, out_hbm, s, c * N_NODES)


def _make_sc_degree():
    return functools.partial(
        pl.kernel,
        out_type=jax.ShapeDtypeStruct((NC * N_NODES, FC), jnp.float32),
        mesh=plsc.VectorSubcoreMesh(**_MESH),
        scratch_types=[
            pltpu.VMEM((NB, K), jnp.int32),
            pltpu.VMEM((K, FC), jnp.float32),
            pltpu.VMEM((ZROWS, FC), jnp.float32),
            pltpu.VMEM_SHARED((N_NODES, FC), jnp.float32),
        ] + [pltpu.SemaphoreType.DMA] * 8,
    )(_sc_degree_body)


def _sc_agg_body(split_edges, src_hbm, dst_hbm, y_hbm, out_hbm,
                 idxg, idxd, rows, zbuf, acc,
                 i0, i1, i2, i3, g0, g1, g2, g3, t0, t1, t2, t3):
    c = lax.axis_index("c")
    s = lax.axis_index("s")
    _fill(zbuf, ZROWS, 0.0)
    _zero_slab(zbuf, acc, s)

    if split_edges:
        e_per = N_EDGES // (NC * NS)
        base = (c * NS + s) * e_per
        off = None
    else:
        e_per = N_EDGES // NS
        base = s * e_per
        off = c * N_NODES
    nch = e_per // K
    isems = (i0, i1, i2, i3)
    gsems = (g0, g1, g2, g3)
    ssems = (t0, t1, t2, t3)

    def fire_i(g, b):
        eb = base + g * K
        pltpu.async_copy(src_hbm.at[pl.ds(eb, K)], idxg.at[b], isems[b])
        pltpu.async_copy(dst_hbm.at[pl.ds(eb, K)], idxd.at[b], isems[b])

    def wait_i(b):
        pltpu.make_async_copy(src_hbm.at[pl.ds(base, K)], idxg.at[b],
                              isems[b]).wait()
        pltpu.make_async_copy(dst_hbm.at[pl.ds(base, K)], idxd.at[b],
                              isems[b]).wait()
        if off is not None:
            for j in range(K // LANES):
                v = idxg[b, pl.ds(j * LANES, LANES)]
                idxg[b, pl.ds(j * LANES, LANES)] = v + off

    def fire_g(g, b):
        pltpu.async_copy(y_hbm.at[idxg.at[b]], rows.at[b], gsems[b])

    def wait_g(b):
        pltpu.make_async_copy(y_hbm.at[idxg.at[b]], rows.at[b],
                              gsems[b]).wait()

    def fire_s(g, b):
        pltpu.async_copy(rows.at[b], acc.at[idxd.at[b]], ssems[b], add=True)

    def wait_s(b):
        pltpu.make_async_copy(rows.at[b], acc.at[idxd.at[b]],
                              ssems[b]).wait()

    def slot(g, j):
        wait_g(j)
        jp = (j + 3) % NB
        _maybe(g - 1 >= 0 if isinstance(g, int) else g >= 1,
               lambda: wait_s(jp))
        fire_s(g, j)
        jn = (j + 2) % NB

        def _adv():
            wait_i(jn)
            fire_g(g + 2, jn)

        _maybe(g + 2 < nch, _adv)
        _maybe(g + 3 < nch, lambda: fire_i(g + 3, jp))

    for g in range(3):
        fire_i(g, g)
    wait_i(0)
    fire_g(0, 0)
    wait_i(1)
    fire_g(1, 1)

    def quad(i, carry):
        for j in range(NB):
            slot(i * NB + j, j)
        return carry

    nquad = nch // NB
    lax.fori_loop(0, nquad, quad, 0)
    for g in range(nquad * NB, nch):
        slot(g, g % NB)
    wait_s((nch - 1) % NB)
    _writeout_slab(acc, out_hbm, s, c * N_NODES)


def _make_sc_agg(split_edges):
    return functools.partial(
        pl.kernel,
        out_type=jax.ShapeDtypeStruct((NC * N_NODES, FC), jnp.float32),
        mesh=plsc.VectorSubcoreMesh(**_MESH),
        scratch_types=[
            pltpu.VMEM((NB, K), jnp.int32),
            pltpu.VMEM((NB, K), jnp.int32),
            pltpu.VMEM((NB, K, FC), jnp.float32),
            pltpu.VMEM((ZROWS, FC), jnp.float32),
            pltpu.VMEM_SHARED((N_NODES, FC), jnp.float32),
        ] + [pltpu.SemaphoreType.DMA] * 12,
    )(functools.partial(_sc_agg_body, split_edges))


def _dinv_from_parts(degp):
    deg = degp[0:N_NODES, 0:1] + degp[N_NODES:2 * N_NODES, 0:1] + 1.0
    return lax.rsqrt(deg)


def _tc_scale0_body(x_ref, degp_ref, z0_ref):
    dinv = _dinv_from_parts(degp_ref[...])
    z0_ref[...] = x_ref[...] * dinv


def _tc_layer2_body(agg1_ref, z0_ref, w1_ref, b1_ref, degp_ref, z_ref):
    dinv = _dinv_from_parts(degp_ref[...])
    agg = agg1_ref[0:N_NODES, :] + agg1_ref[N_NODES:2 * N_NODES, :]
    m1 = (agg + z0_ref[...]) * dinv
    h = jnp.maximum(
        jnp.dot(m1, w1_ref[...], preferred_element_type=jnp.float32)
        + b1_ref[...], 0.0)
    z_ref[...] = h * dinv


def _tc_final_body(agg2_ref, z_ref, w2_ref, b2_ref, degp_ref, batch_ref,
                   wg_ref, bg_ref, emb_ref, logit_ref):
    dinv = _dinv_from_parts(degp_ref[...])
    agg = agg2_ref[0:N_NODES, :] + agg2_ref[N_NODES:2 * N_NODES, :]
    m = (agg + z_ref[...]) * dinv
    ngr = emb_ref.shape[0]
    seg = batch_ref[...]
    p = (seg == lax.broadcasted_iota(jnp.int32, (1, ngr), 1)).astype(
        jnp.float32)
    pooled = lax.dot_general(p, m, (((0,), (0,)), ((), ())),
                             preferred_element_type=jnp.float32)
    counts = lax.dot_general(p, jnp.ones((N_NODES, 1), jnp.float32),
                             (((0,), (0,)), ((), ())),
                             preferred_element_type=jnp.float32)
    sums = (jnp.dot(pooled, w2_ref[...], preferred_element_type=jnp.float32)
            + counts * b2_ref[...])
    emb = sums / jnp.maximum(counts, 1.0)
    emb_ref[...] = emb
    logit_ref[...] = (jnp.dot(emb, wg_ref[...],
                              preferred_element_type=jnp.float32)
                      + bg_ref[...])


def kernel(x, edge_index, batch, W1, b1, W2, b2, Wg, bg):
    n, _ = x.shape
    hid = W1.shape[1]
    emb_d = W2.shape[1]
    ngroups = Wg.shape[1]
    ngraphs = 64
    src = edge_index[0]
    dst = edge_index[1]

    degp = _make_sc_degree()(dst)

    z0 = pl.pallas_call(
        _tc_scale0_body,
        out_shape=jax.ShapeDtypeStruct((n, x.shape[1]), jnp.float32),
    )(x, degp)

    agg1 = _make_sc_agg(split_edges=True)(src, dst, z0)

    z = pl.pallas_call(
        _tc_layer2_body,
        out_shape=jax.ShapeDtypeStruct((n, hid), jnp.float32),
    )(agg1, z0, W1, b1.reshape(1, -1), degp)

    agg2 = _make_sc_agg(split_edges=True)(src, dst, z)

    emb, logits = pl.pallas_call(
        _tc_final_body,
        out_shape=(
            jax.ShapeDtypeStruct((ngraphs, emb_d), jnp.float32),
            jax.ShapeDtypeStruct((ngraphs, ngroups), jnp.float32),
        ),
    )(agg2, z, W2, b2.reshape(1, -1), degp, batch.reshape(-1, 1), Wg,
      bg.reshape(1, -1))

    return emb, logits
